# baseline (speedup 1.0000x reference)
import functools

import jax
import jax.numpy as jnp
from jax import lax
from jax.experimental import pallas as pl
from jax.experimental.pallas import tpu as pltpu
from jax.experimental.pallas import tpu_sc as plsc

NUM_USERS = 1000000
EMBED_DIM = 64
BATCH = 16384

_info = plsc.get_sparse_core_info()
_NC, _NS, _L = _info.num_cores, _info.num_subcores, _info.num_lanes
_NW = _NC * _NS
_BPW = BATCH // _NW
_GROUPS = EMBED_DIM // _L

_mesh = plsc.VectorSubcoreMesh(core_axis_name="c", subcore_axis_name="s")


@functools.partial(
    pl.kernel,
    mesh=_mesh,
    out_type=jax.ShapeDtypeStruct((BATCH, EMBED_DIM), jnp.float32),
    scratch_types=[
        pltpu.VMEM((_BPW,), jnp.int32),
        pltpu.VMEM((_BPW, EMBED_DIM), jnp.float32),
        pltpu.VMEM((EMBED_DIM,), jnp.float32),
        pltpu.SemaphoreType.DMA,
    ],
    compiler_params=pltpu.CompilerParams(use_tc_tiling_on_sc=False),
)
def _sc_lookup(table_hbm, idx_hbm, rel_hbm, out_hbm, idx_v, rows_v, rel_v, sem):
    wid = lax.axis_index("s") * _NC + lax.axis_index("c")
    base = wid * _BPW

    pltpu.sync_copy(idx_hbm.at[pl.ds(base, _BPW)], idx_v)
    pltpu.sync_copy(rel_hbm, rel_v)
    gather = pltpu.async_copy(table_hbm.at[idx_v], rows_v, sem)

    rel_regs = [rel_v[pl.ds(g * _L, _L)] for g in range(_GROUPS)]
    gather.wait()

    def add_row(i, carry):
        for g in range(_GROUPS):
            sl = pl.ds(g * _L, _L)
            rows_v[i, sl] = rows_v[i, sl] + rel_regs[g]
        return carry

    lax.fori_loop(0, _BPW, add_row, 0, unroll=4)

    pltpu.sync_copy(rows_v, out_hbm.at[pl.ds(base, _BPW)])


def kernel(batch_users, user_emb_weight, rel_emb_weight):
    idx = batch_users.astype(jnp.int32)
    rel_row = rel_emb_weight[-1]
    return _sc_lookup(user_emb_weight, idx, rel_row)


# speedup vs baseline: 1.7067x; 1.7067x over previous
import functools

import jax
import jax.numpy as jnp
from jax import lax
from jax.experimental import pallas as pl
from jax.experimental.pallas import tpu as pltpu
from jax.experimental.pallas import tpu_sc as plsc

NUM_USERS = 1000000
EMBED_DIM = 64
BATCH = 16384

_info = plsc.get_sparse_core_info()
_NC, _NS, _L = _info.num_cores, _info.num_subcores, _info.num_lanes
_NW = _NC * _NS
_BPW = BATCH // _NW
_GROUPS = EMBED_DIM // _L
_CHUNK = 128

_mesh = plsc.VectorSubcoreMesh(core_axis_name="c", subcore_axis_name="s")


@functools.partial(
    pl.kernel,
    mesh=_mesh,
    out_type=jax.ShapeDtypeStruct((BATCH, EMBED_DIM), jnp.float32),
    scratch_types=[
        pltpu.VMEM((_BPW,), jnp.int32),
        pltpu.SMEM((_BPW,), jnp.int32),
        pltpu.VMEM((_BPW, EMBED_DIM), jnp.float32),
        pltpu.VMEM((EMBED_DIM,), jnp.float32),
        pltpu.SemaphoreType.DMA,
        pltpu.SemaphoreType.DMA,
    ],
)
def _sc_lookup(table_hbm, idx_hbm, rel_hbm, out_hbm,
               idx_v, idx_s, rows_v, rel_v, sem, row_sem):
    wid = lax.axis_index("s") * _NC + lax.axis_index("c")
    base = wid * _BPW

    pltpu.sync_copy(idx_hbm.at[pl.ds(base, _BPW)], idx_v)
    pltpu.sync_copy(rel_hbm, rel_v)

    def fire_vec(v, carry):
        vec = idx_v[pl.ds(v * _L, _L)]
        for j in range(_L):
            r = vec[j]
            pltpu.make_async_copy(
                table_hbm.at[pl.ds(r, 1)],
                rows_v.at[pl.ds(v * _L + j, 1)],
                row_sem,
            ).start()
        return carry

    def burst(c, carry):
        lax.fori_loop(c * (_CHUNK // _L), (c + 1) * (_CHUNK // _L), fire_vec, 0)
        pltpu.make_async_copy(
            table_hbm.at[pl.ds(0, _CHUNK)],
            rows_v.at[pl.ds(c * _CHUNK, _CHUNK)],
            row_sem,
        ).wait()
        return carry

    lax.fori_loop(0, _BPW // _CHUNK, burst, 0)

    rel_regs = [rel_v[pl.ds(g * _L, _L)] for g in range(_GROUPS)]

    def add_row(i, carry):
        for g in range(_GROUPS):
            sl = pl.ds(g * _L, _L)
            rows_v[i, sl] = rows_v[i, sl] + rel_regs[g]
        return carry

    lax.fori_loop(0, _BPW, add_row, 0, unroll=4)

    pltpu.sync_copy(rows_v, out_hbm.at[pl.ds(base, _BPW)])


def kernel(batch_users, user_emb_weight, rel_emb_weight):
    idx = batch_users.astype(jnp.int32)
    rel_row = rel_emb_weight[-1]
    return _sc_lookup(user_emb_weight, idx, rel_row)


# speedup vs baseline: 1.7067x; 1.0000x over previous
import functools

import jax
import jax.numpy as jnp
from jax import lax
from jax.experimental import pallas as pl
from jax.experimental.pallas import tpu as pltpu
from jax.experimental.pallas import tpu_sc as plsc

NUM_USERS = 1000000
EMBED_DIM = 64
BATCH = 16384

_info = plsc.get_sparse_core_info()
_NC, _NS, _L = _info.num_cores, _info.num_subcores, _info.num_lanes
_NW = _NC * _NS
_BPW = BATCH // _NW
_GROUPS = EMBED_DIM // _L
_CHUNK = 128

_mesh = plsc.VectorSubcoreMesh(core_axis_name="c", subcore_axis_name="s")


@functools.partial(
    pl.kernel,
    mesh=_mesh,
    out_type=jax.ShapeDtypeStruct((BATCH, EMBED_DIM), jnp.float32),
    scratch_types=[
        pltpu.VMEM((_BPW,), jnp.int32),
        pltpu.VMEM((_BPW, EMBED_DIM), jnp.float32),
        pltpu.VMEM((EMBED_DIM,), jnp.float32),
        pltpu.SemaphoreType.DMA,
        pltpu.SemaphoreType.DMA,
    ],
)
def _sc_lookup(table_hbm, idx_hbm, rel_hbm, out_hbm,
               idx_v, rows_v, rel_v, sem, row_sem):
    wid = lax.axis_index("s") * _NC + lax.axis_index("c")
    base = wid * _BPW

    pltpu.sync_copy(idx_hbm.at[pl.ds(base, _BPW)], idx_v)
    pltpu.sync_copy(rel_hbm, rel_v)

    def fire_vec(v, carry):
        vec = idx_v[pl.ds(v * _L, _L)]
        for j in range(_L):
            r = vec[j]
            pltpu.make_async_copy(
                table_hbm.at[pl.ds(r, 1)],
                rows_v.at[pl.ds(v * _L + j, 1)],
                row_sem,
            ).start()
        return carry

    def burst(c, carry):
        lax.fori_loop(c * (_CHUNK // _L), (c + 1) * (_CHUNK // _L), fire_vec, 0)
        pltpu.make_async_copy(
            table_hbm.at[pl.ds(0, _CHUNK)],
            rows_v.at[pl.ds(c * _CHUNK, _CHUNK)],
            row_sem,
        ).wait()
        return carry

    lax.fori_loop(0, _BPW // _CHUNK, burst, 0)

    rel_regs = [rel_v[pl.ds(g * _L, _L)] for g in range(_GROUPS)]

    def add_row(i, carry):
        for g in range(_GROUPS):
            sl = pl.ds(g * _L, _L)
            rows_v[i, sl] = rows_v[i, sl] + rel_regs[g]
        return carry

    lax.fori_loop(0, _BPW, add_row, 0, unroll=4)

    pltpu.sync_copy(rows_v, out_hbm.at[pl.ds(base, _BPW)])


def kernel(batch_users, user_emb_weight, rel_emb_weight):
    idx = batch_users.astype(jnp.int32)
    rel_row = rel_emb_weight[-1]
    return _sc_lookup(user_emb_weight, idx, rel_row)


# speedup vs baseline: 2.5038x; 1.4670x over previous
import functools

import jax
import jax.numpy as jnp
from jax import lax
from jax.experimental import pallas as pl
from jax.experimental.pallas import tpu as pltpu
from jax.experimental.pallas import tpu_sc as plsc

NUM_USERS = 1000000
EMBED_DIM = 64
BATCH = 16384
_BLK = 128

_info = plsc.get_sparse_core_info()
_NC, _NS, _L = _info.num_cores, _info.num_subcores, _info.num_lanes
_NW = _NC * _NS
_BPW = BATCH // _NW
_NV = _BPW // _L
_GROUPS = EMBED_DIM // _L
_NBUF = 4

_mesh = plsc.VectorSubcoreMesh(core_axis_name="c", subcore_axis_name="s")


@functools.partial(
    pl.kernel,
    mesh=_mesh,
    out_type=jax.ShapeDtypeStruct((BATCH, EMBED_DIM), jnp.float32),
    scratch_types=[
        pltpu.VMEM((_BPW + _L,), jnp.int32),
        pltpu.VMEM((_BPW, EMBED_DIM), jnp.float32),
        pltpu.VMEM((EMBED_DIM,), jnp.float32),
    ]
    + [pltpu.VMEM((EMBED_DIM, _BLK), jnp.float32) for _ in range(_NBUF)]
    + [pltpu.SemaphoreType.DMA for _ in range(_NBUF)]
    + [pltpu.SemaphoreType.DMA],
    compiler_params=pltpu.CompilerParams(needs_layout_passes=False),
)
def _sc_lookup(table_t_hbm, idx_hbm, rel_hbm, out_hbm,
               idx_v, rows_v, rel_v,
               b0, b1, b2, b3, s0, s1, s2, s3, sem):
    bufs = (b0, b1, b2, b3)
    sems = (s0, s1, s2, s3)
    wid = lax.axis_index("s") * _NC + lax.axis_index("c")
    base = wid * _BPW

    pltpu.sync_copy(idx_hbm.at[pl.ds(base, _BPW)], idx_v.at[pl.ds(0, _BPW)])
    pltpu.sync_copy(idx_hbm.at[pl.ds(0, _L)], idx_v.at[pl.ds(_BPW, _L)])
    pltpu.sync_copy(rel_hbm, rel_v)
    rel_regs = [rel_v[pl.ds(g * _L, _L)] for g in range(_GROUPS)]
    c0 = lax.iota(jnp.int32, _L)
    ones = c0 * 0 + 1

    def fire(slot, u):
        u = lax.min(lax.max(u, 0), NUM_USERS - 1)
        bstart = pl.multiple_of((u // _BLK) * _BLK, _BLK)
        pltpu.make_async_copy(
            table_t_hbm.at[:, pl.ds(bstart, _BLK)], bufs[slot], sems[slot]
        ).start()

    vec0 = idx_v[pl.ds(0, _L)]
    for j in range(_NBUF):
        fire(j, vec0[j])

    def body(v, carry):
        vec = idx_v[pl.ds(v * _L, _L)]
        vecn = idx_v[pl.ds(v * _L + _L, _L)]
        for j in range(_L):
            i = v * _L + j
            slot = j % _NBUF
            pltpu.make_async_copy(
                table_t_hbm.at[:, pl.ds(0, _BLK)], bufs[slot], sems[slot]
            ).wait()
            u = vec[j]
            colv = ones * (u % _BLK)
            for g in range(_GROUPS):
                vals = plsc.load_gather(bufs[slot], [c0 + g * _L, colv])
                rows_v[i, pl.ds(g * _L, _L)] = vals + rel_regs[g]
            u_next = vec[j + _NBUF] if j + _NBUF < _L else vecn[j + _NBUF - _L]
            fire(slot, u_next)
        return carry

    lax.fori_loop(0, _NV, body, 0)

    for j in range(_NBUF):
        pltpu.make_async_copy(
            table_t_hbm.at[:, pl.ds(0, _BLK)], bufs[j], sems[j]
        ).wait()

    pltpu.sync_copy(rows_v, out_hbm.at[pl.ds(base, _BPW)])


def kernel(batch_users, user_emb_weight, rel_emb_weight):
    idx = batch_users.astype(jnp.int32)
    table_t = jnp.swapaxes(user_emb_weight, 0, 1)
    rel_row = rel_emb_weight[-1]
    return _sc_lookup(table_t, idx, rel_row)


# speedup vs baseline: 2.9331x; 1.1715x over previous
import functools

import jax
import jax.numpy as jnp
from jax import lax
from jax.experimental import pallas as pl
from jax.experimental.pallas import tpu as pltpu
from jax.experimental.pallas import tpu_sc as plsc

NUM_USERS = 1000000
EMBED_DIM = 64
BATCH = 16384
_BLK = 128

_info = plsc.get_sparse_core_info()
_NC, _NS, _L = _info.num_cores, _info.num_subcores, _info.num_lanes
_NW = _NC * _NS
_BPW = BATCH // _NW
_NV = _BPW // _L
_GROUPS = EMBED_DIM // _L
_NBUF = 8

_mesh = plsc.VectorSubcoreMesh(core_axis_name="c", subcore_axis_name="s")


@functools.partial(
    pl.kernel,
    mesh=_mesh,
    out_type=jax.ShapeDtypeStruct((BATCH, EMBED_DIM), jnp.float32),
    scratch_types=[
        pltpu.VMEM((_BPW + _L,), jnp.int32),
        pltpu.VMEM((4 * _L, EMBED_DIM), jnp.float32),
        pltpu.VMEM((EMBED_DIM,), jnp.float32),
    ]
    + [pltpu.VMEM((EMBED_DIM, _BLK), jnp.float32) for _ in range(_NBUF)]
    + [pltpu.SemaphoreType.DMA for _ in range(_NBUF)]
    + [pltpu.SemaphoreType.DMA],
    compiler_params=pltpu.CompilerParams(needs_layout_passes=False),
)
def _sc_lookup(table_t_hbm, idx_hbm, rel_hbm, out_hbm,
               idx_v, rows_v, rel_v,
               b0, b1, b2, b3, b4, b5, b6, b7,
               s0, s1, s2, s3, s4, s5, s6, s7, sem):
    bufs = (b0, b1, b2, b3, b4, b5, b6, b7)
    sems = (s0, s1, s2, s3, s4, s5, s6, s7)
    wid = lax.axis_index("s") * _NC + lax.axis_index("c")
    base = wid * _BPW

    pltpu.sync_copy(idx_hbm.at[pl.ds(base, _BPW)], idx_v.at[pl.ds(0, _BPW)])
    pltpu.sync_copy(idx_hbm.at[pl.ds(0, _L)], idx_v.at[pl.ds(_BPW, _L)])
    pltpu.sync_copy(rel_hbm, rel_v)
    rel_regs = [rel_v[pl.ds(g * _L, _L)] for g in range(_GROUPS)]
    c0 = lax.iota(jnp.int32, _L)
    ones = c0 * 0 + 1

    def fire(slot, u):
        u = lax.min(lax.max(u, 0), NUM_USERS - 1)
        bstart = pl.multiple_of((u // _BLK) * _BLK, _BLK)
        pltpu.make_async_copy(
            table_t_hbm.at[:, pl.ds(bstart, _BLK)], bufs[slot], sems[slot]
        ).start()

    vec0 = idx_v[pl.ds(0, _L)]
    for j in range(_NBUF):
        fire(j, vec0[j])

    def body(v, carry):
        vec = idx_v[pl.ds(v * _L, _L)]
        vecn = idx_v[pl.ds(v * _L + _L, _L)]
        bank = lax.rem(v, 4) * _L
        @pl.when(v >= 4)
        def _():
            pltpu.make_async_copy(
                rows_v.at[pl.ds(0, _L)],
                out_hbm.at[pl.ds(base, _L)],
                sem,
            ).wait()
        for j in range(_L):
            i = v * _L + j
            slot = j % _NBUF
            pltpu.make_async_copy(
                table_t_hbm.at[:, pl.ds(0, _BLK)], bufs[slot], sems[slot]
            ).wait()
            u = vec[j]
            colv = ones * (u % _BLK)
            for g in range(_GROUPS):
                vals = plsc.load_gather(bufs[slot], [c0 + g * _L, colv])
                rows_v[bank + j, pl.ds(g * _L, _L)] = vals + rel_regs[g]
            u_next = vec[j + _NBUF] if j + _NBUF < _L else vecn[j + _NBUF - _L]
            fire(slot, u_next)
        pltpu.make_async_copy(
            rows_v.at[pl.ds(bank, _L)],
            out_hbm.at[pl.ds(base + v * _L, _L)],
            sem,
        ).start()
        return carry

    lax.fori_loop(0, _NV, body, 0)

    for j in range(_NBUF):
        pltpu.make_async_copy(
            table_t_hbm.at[:, pl.ds(0, _BLK)], bufs[j], sems[j]
        ).wait()
    for _ in range(4):
        pltpu.make_async_copy(
            rows_v.at[pl.ds(0, _L)], out_hbm.at[pl.ds(base, _L)], sem
        ).wait()


def kernel(batch_users, user_emb_weight, rel_emb_weight):
    idx = batch_users.astype(jnp.int32)
    table_t = jnp.swapaxes(user_emb_weight, 0, 1)
    rel_row = rel_emb_weight[-1]
    return _sc_lookup(table_t, idx, rel_row)
